# initial kernel scaffold (unmeasured)
import jax
import jax.numpy as jnp
from jax import lax
from jax.experimental import pallas as pl
from jax.experimental.pallas import tpu as pltpu

M = 2048
D = 2048
HALF = M // 2


def kernel(partial, resid, gamma):
    partial2 = partial.reshape(M, D)
    gamma2 = gamma.reshape(1, D)

    def body(
        p_hbm,
        r_hbm,
        g_ref,
        out_hbm,
        p_mine,
        r_mine,
        x_send,
        x_recv,
        y_send,
        y_recv,
        out_mine,
        local_sems,
        send_sems,
        recv_sems,
    ):
        my_x = lax.axis_index("x")
        my_y = lax.axis_index("y")
        row0 = my_y * HALF
        other_row0 = (1 - my_y) * HALF

        cp_p = pltpu.make_async_copy(
            p_hbm.at[pl.ds(row0, HALF), :], p_mine, local_sems.at[0]
        )
        cp_r = pltpu.make_async_copy(
            r_hbm.at[pl.ds(row0, HALF), :], r_mine, local_sems.at[1]
        )
        cp_p.start()
        cp_r.start()
        cp_p.wait()
        x_send[...] = p_mine[...].astype(jnp.bfloat16)

        rdma_x = pltpu.make_async_remote_copy(
            src_ref=x_send,
            dst_ref=x_recv,
            send_sem=send_sems.at[0],
            recv_sem=recv_sems.at[0],
            device_id=(1 - my_x, my_y),
            device_id_type=pl.DeviceIdType.MESH,
        )
        rdma_x.start()
        rdma_x.wait()
        cp_r.wait()

        t = p_mine[...] + x_recv[...].astype(jnp.float32) + r_mine[...]
        rms = jnp.sqrt(jnp.mean(t * t, axis=-1, keepdims=True) + 1e-6)
        out_mine[...] = t / rms * g_ref[...]

        cp_out = pltpu.make_async_copy(
            out_mine, out_hbm.at[pl.ds(row0, HALF), :], local_sems.at[0]
        )
        cp_out.start()
        y_send[...] = out_mine[...].astype(jnp.bfloat16)

        rdma_y = pltpu.make_async_remote_copy(
            src_ref=y_send,
            dst_ref=y_recv,
            send_sem=send_sems.at[1],
            recv_sem=recv_sems.at[1],
            device_id=(my_x, 1 - my_y),
            device_id_type=pl.DeviceIdType.MESH,
        )
        rdma_y.start()
        rdma_y.wait()

        p_mine[...] = y_recv[...].astype(jnp.float32)
        cp_out2 = pltpu.make_async_copy(
            p_mine, out_hbm.at[pl.ds(other_row0, HALF), :], local_sems.at[1]
        )
        cp_out2.start()
        cp_out.wait()
        cp_out2.wait()

    return pl.pallas_call(
        body,
        out_shape=jax.ShapeDtypeStruct((M, D), jnp.float32),
        in_specs=[
            pl.BlockSpec(memory_space=pltpu.ANY),
            pl.BlockSpec(memory_space=pltpu.ANY),
            pl.BlockSpec(memory_space=pltpu.VMEM),
        ],
        out_specs=pl.BlockSpec(memory_space=pltpu.ANY),
        scratch_shapes=[
            pltpu.VMEM((HALF, D), jnp.float32),
            pltpu.VMEM((HALF, D), jnp.float32),
            pltpu.VMEM((HALF, D), jnp.bfloat16),
            pltpu.VMEM((HALF, D), jnp.bfloat16),
            pltpu.VMEM((HALF, D), jnp.bfloat16),
            pltpu.VMEM((HALF, D), jnp.bfloat16),
            pltpu.VMEM((HALF, D), jnp.float32),
            pltpu.SemaphoreType.DMA((2,)),
            pltpu.SemaphoreType.DMA((2,)),
            pltpu.SemaphoreType.DMA((2,)),
        ],
    )(partial2, resid, gamma2)


# baseline (device time: 123511 ns/iter reference)
import jax
import jax.numpy as jnp
from jax import lax
from jax.experimental import pallas as pl
from jax.experimental.pallas import tpu as pltpu

M = 2048
D = 2048
HALF = M // 2


def kernel(partial, resid, gamma):
    partial2 = partial.reshape(M, D)
    gamma2 = gamma.reshape(1, D)

    def body(
        p_hbm,
        r_hbm,
        g_ref,
        out_hbm,
        p_mine,
        r_mine,
        x_send,
        x_recv,
        y_send,
        y_recv,
        out_mine,
        local_sems,
        send_sems,
        recv_sems,
    ):
        my_x = lax.axis_index("x")
        my_y = lax.axis_index("y")
        row0 = my_y * HALF
        other_row0 = (1 - my_y) * HALF

        cp_p = pltpu.make_async_copy(
            p_hbm.at[pl.ds(row0, HALF), :], p_mine, local_sems.at[0]
        )
        cp_r = pltpu.make_async_copy(
            r_hbm.at[pl.ds(row0, HALF), :], r_mine, local_sems.at[1]
        )
        cp_p.start()
        cp_r.start()
        cp_p.wait()
        x_send[...] = p_mine[...].astype(jnp.bfloat16)

        rdma_x = pltpu.make_async_remote_copy(
            src_ref=x_send,
            dst_ref=x_recv,
            send_sem=send_sems.at[0],
            recv_sem=recv_sems.at[0],
            device_id=(1 - my_x, my_y),
            device_id_type=pl.DeviceIdType.MESH,
        )
        rdma_x.start()
        rdma_x.wait()
        cp_r.wait()

        t = p_mine[...] + x_recv[...].astype(jnp.float32) + r_mine[...]
        rms = jnp.sqrt(jnp.mean(t * t, axis=-1, keepdims=True) + 1e-6)
        out_mine[...] = t / rms * g_ref[...]

        cp_out = pltpu.make_async_copy(
            out_mine, out_hbm.at[pl.ds(row0, HALF), :], local_sems.at[0]
        )
        cp_out.start()
        y_send[...] = out_mine[...].astype(jnp.bfloat16)

        rdma_y = pltpu.make_async_remote_copy(
            src_ref=y_send,
            dst_ref=y_recv,
            send_sem=send_sems.at[1],
            recv_sem=recv_sems.at[1],
            device_id=(my_x, 1 - my_y),
            device_id_type=pl.DeviceIdType.MESH,
        )
        rdma_y.start()
        rdma_y.wait()

        p_mine[...] = y_recv[...].astype(jnp.float32)
        cp_out2 = pltpu.make_async_copy(
            p_mine, out_hbm.at[pl.ds(other_row0, HALF), :], local_sems.at[1]
        )
        cp_out2.start()
        cp_out.wait()
        cp_out2.wait()

    return pl.pallas_call(
        body,
        out_shape=jax.ShapeDtypeStruct((M, D), jnp.float32),
        in_specs=[
            pl.BlockSpec(memory_space=pl.ANY),
            pl.BlockSpec(memory_space=pl.ANY),
            pl.BlockSpec(memory_space=pltpu.VMEM),
        ],
        out_specs=pl.BlockSpec(memory_space=pl.ANY),
        scratch_shapes=[
            pltpu.VMEM((HALF, D), jnp.float32),
            pltpu.VMEM((HALF, D), jnp.float32),
            pltpu.VMEM((HALF, D), jnp.bfloat16),
            pltpu.VMEM((HALF, D), jnp.bfloat16),
            pltpu.VMEM((HALF, D), jnp.bfloat16),
            pltpu.VMEM((HALF, D), jnp.bfloat16),
            pltpu.VMEM((HALF, D), jnp.float32),
            pltpu.SemaphoreType.DMA((2,)),
            pltpu.SemaphoreType.DMA((2,)),
            pltpu.SemaphoreType.DMA((2,)),
        ],
        compiler_params=pltpu.CompilerParams(
            vmem_limit_bytes=100 * 1024 * 1024,
        ),
    )(partial2, resid, gamma2)


# device time: 79242 ns/iter; 1.5587x vs baseline; 1.5587x over previous
import jax
import jax.numpy as jnp
from jax import lax
from jax.experimental import pallas as pl
from jax.experimental.pallas import tpu as pltpu

M = 2048
D = 2048
HALF = M // 2
NC = 8
ROWS = HALF // NC


def kernel(partial, resid, gamma):
    partial2 = partial.reshape(M, D)
    gamma2 = gamma.reshape(1, D)

    def body(
        p_hbm,
        r_hbm,
        g_ref,
        out_hbm,
        p_mine,
        r_mine,
        x_send,
        x_recv,
        y_send,
        y_recv,
        out_mine,
        load_sems,
        store_sem,
        x_send_sems,
        x_recv_sems,
        y_send_sems,
        y_recv_sems,
    ):
        my_x = lax.axis_index("x")
        my_y = lax.axis_index("y")
        row0 = my_y * HALF
        other_row0 = (1 - my_y) * HALF

        cp_p = pltpu.make_async_copy(
            p_hbm.at[pl.ds(row0, HALF), :], p_mine, load_sems.at[0]
        )
        cp_r = pltpu.make_async_copy(
            r_hbm.at[pl.ds(row0, HALF), :], r_mine, load_sems.at[1]
        )
        cp_p.start()
        cp_r.start()
        cp_p.wait()
        x_send[...] = p_mine[...].astype(jnp.bfloat16)

        def chunk(ref, c):
            return ref.at[pl.ds(c * ROWS, ROWS), :]

        x_rdmas = []
        for c in range(NC):
            rdma = pltpu.make_async_remote_copy(
                src_ref=chunk(x_send, c),
                dst_ref=chunk(x_recv, c),
                send_sem=x_send_sems.at[c],
                recv_sem=x_recv_sems.at[c],
                device_id=(1 - my_x, my_y),
                device_id_type=pl.DeviceIdType.MESH,
            )
            rdma.start()
            x_rdmas.append(rdma)

        cp_r.wait()

        y_rdmas = []
        stores = []
        for c in range(NC):
            x_rdmas[c].wait_recv()
            sl = pl.ds(c * ROWS, ROWS)
            t = (
                p_mine[sl, :]
                + x_recv[sl, :].astype(jnp.float32)
                + r_mine[sl, :]
            )
            rms = jnp.sqrt(jnp.mean(t * t, axis=-1, keepdims=True) + 1e-6)
            out_mine[sl, :] = t / rms * g_ref[...]
            y_send[sl, :] = out_mine[sl, :].astype(jnp.bfloat16)
            rdma = pltpu.make_async_remote_copy(
                src_ref=chunk(y_send, c),
                dst_ref=chunk(y_recv, c),
                send_sem=y_send_sems.at[c],
                recv_sem=y_recv_sems.at[c],
                device_id=(my_x, 1 - my_y),
                device_id_type=pl.DeviceIdType.MESH,
            )
            rdma.start()
            y_rdmas.append(rdma)
            cp = pltpu.make_async_copy(
                chunk(out_mine, c),
                out_hbm.at[pl.ds(row0 + c * ROWS, ROWS), :],
                store_sem,
            )
            cp.start()
            stores.append(cp)

        for c in range(NC):
            y_rdmas[c].wait_recv()
            sl = pl.ds(c * ROWS, ROWS)
            p_mine[sl, :] = y_recv[sl, :].astype(jnp.float32)
            cp = pltpu.make_async_copy(
                chunk(p_mine, c),
                out_hbm.at[pl.ds(other_row0 + c * ROWS, ROWS), :],
                store_sem,
            )
            cp.start()
            stores.append(cp)

        for c in range(NC):
            x_rdmas[c].wait_send()
            y_rdmas[c].wait_send()
        for cp in stores:
            cp.wait()

    return pl.pallas_call(
        body,
        out_shape=jax.ShapeDtypeStruct((M, D), jnp.float32),
        in_specs=[
            pl.BlockSpec(memory_space=pl.ANY),
            pl.BlockSpec(memory_space=pl.ANY),
            pl.BlockSpec(memory_space=pltpu.VMEM),
        ],
        out_specs=pl.BlockSpec(memory_space=pl.ANY),
        scratch_shapes=[
            pltpu.VMEM((HALF, D), jnp.float32),
            pltpu.VMEM((HALF, D), jnp.float32),
            pltpu.VMEM((HALF, D), jnp.bfloat16),
            pltpu.VMEM((HALF, D), jnp.bfloat16),
            pltpu.VMEM((HALF, D), jnp.bfloat16),
            pltpu.VMEM((HALF, D), jnp.bfloat16),
            pltpu.VMEM((HALF, D), jnp.float32),
            pltpu.SemaphoreType.DMA((2,)),
            pltpu.SemaphoreType.DMA,
            pltpu.SemaphoreType.DMA((NC,)),
            pltpu.SemaphoreType.DMA((NC,)),
            pltpu.SemaphoreType.DMA((NC,)),
            pltpu.SemaphoreType.DMA((NC,)),
        ],
        compiler_params=pltpu.CompilerParams(
            vmem_limit_bytes=100 * 1024 * 1024,
        ),
    )(partial2, resid, gamma2)


# device time: 73621 ns/iter; 1.6777x vs baseline; 1.0764x over previous
import jax
import jax.numpy as jnp
from jax import lax
from jax.experimental import pallas as pl
from jax.experimental.pallas import tpu as pltpu

M = 2048
D = 2048
HALF = M // 2
NC = 16
ROWS = HALF // NC
LAG = 3


def kernel(partial, resid, gamma):
    partial2 = partial.reshape(M, D)
    gamma2 = gamma.reshape(1, D)

    def body(
        p_hbm,
        r_hbm,
        g_ref,
        out_hbm,
        p_mine,
        r_mine,
        x_send,
        x_recv,
        y_send,
        y_recv,
        out_mine,
        load_p_sems,
        load_r_sem,
        store_sem,
        x_send_sems,
        x_recv_sems,
        y_send_sems,
        y_recv_sems,
    ):
        my_x = lax.axis_index("x")
        my_y = lax.axis_index("y")
        row0 = my_y * HALF
        other_row0 = (1 - my_y) * HALF

        def chunk(ref, c):
            return ref.at[pl.ds(c * ROWS, ROWS), :]

        loads = []
        for c in range(NC):
            cp = pltpu.make_async_copy(
                p_hbm.at[pl.ds(row0 + c * ROWS, ROWS), :],
                chunk(p_mine, c),
                load_p_sems.at[c],
            )
            cp.start()
            loads.append(cp)
        cp_r = pltpu.make_async_copy(
            r_hbm.at[pl.ds(row0, HALF), :], r_mine, load_r_sem
        )
        cp_r.start()

        x_rdmas = []
        for c in range(NC):
            loads[c].wait()
            x_send[pl.ds(c * ROWS, ROWS), :] = p_mine[
                pl.ds(c * ROWS, ROWS), :
            ].astype(jnp.bfloat16)
            rdma = pltpu.make_async_remote_copy(
                src_ref=chunk(x_send, c),
                dst_ref=chunk(x_recv, c),
                send_sem=x_send_sems.at[c],
                recv_sem=x_recv_sems.at[c],
                device_id=(1 - my_x, my_y),
                device_id_type=pl.DeviceIdType.MESH,
            )
            rdma.start()
            x_rdmas.append(rdma)

        cp_r.wait()

        y_rdmas = []
        stores = []

        def drain_y(c):
            y_rdmas[c].wait_recv()
            sl = pl.ds(c * ROWS, ROWS)
            p_mine[sl, :] = y_recv[sl, :].astype(jnp.float32)
            cp = pltpu.make_async_copy(
                chunk(p_mine, c),
                out_hbm.at[pl.ds(other_row0 + c * ROWS, ROWS), :],
                store_sem,
            )
            cp.start()
            stores.append(cp)

        for c in range(NC):
            x_rdmas[c].wait_recv()
            sl = pl.ds(c * ROWS, ROWS)
            t = (
                p_mine[sl, :]
                + x_recv[sl, :].astype(jnp.float32)
                + r_mine[sl, :]
            )
            rms = jnp.sqrt(jnp.mean(t * t, axis=-1, keepdims=True) + 1e-6)
            out_mine[sl, :] = t / rms * g_ref[...]
            y_send[sl, :] = out_mine[sl, :].astype(jnp.bfloat16)
            rdma = pltpu.make_async_remote_copy(
                src_ref=chunk(y_send, c),
                dst_ref=chunk(y_recv, c),
                send_sem=y_send_sems.at[c],
                recv_sem=y_recv_sems.at[c],
                device_id=(my_x, 1 - my_y),
                device_id_type=pl.DeviceIdType.MESH,
            )
            rdma.start()
            y_rdmas.append(rdma)
            cp = pltpu.make_async_copy(
                chunk(out_mine, c),
                out_hbm.at[pl.ds(row0 + c * ROWS, ROWS), :],
                store_sem,
            )
            cp.start()
            stores.append(cp)
            if c >= LAG:
                drain_y(c - LAG)

        for c in range(NC - LAG, NC):
            drain_y(c)

        for c in range(NC):
            x_rdmas[c].wait_send()
            y_rdmas[c].wait_send()
        for cp in stores:
            cp.wait()

    return pl.pallas_call(
        body,
        out_shape=jax.ShapeDtypeStruct((M, D), jnp.float32),
        in_specs=[
            pl.BlockSpec(memory_space=pl.ANY),
            pl.BlockSpec(memory_space=pl.ANY),
            pl.BlockSpec(memory_space=pltpu.VMEM),
        ],
        out_specs=pl.BlockSpec(memory_space=pl.ANY),
        scratch_shapes=[
            pltpu.VMEM((HALF, D), jnp.float32),
            pltpu.VMEM((HALF, D), jnp.float32),
            pltpu.VMEM((HALF, D), jnp.bfloat16),
            pltpu.VMEM((HALF, D), jnp.bfloat16),
            pltpu.VMEM((HALF, D), jnp.bfloat16),
            pltpu.VMEM((HALF, D), jnp.bfloat16),
            pltpu.VMEM((HALF, D), jnp.float32),
            pltpu.SemaphoreType.DMA((NC,)),
            pltpu.SemaphoreType.DMA,
            pltpu.SemaphoreType.DMA,
            pltpu.SemaphoreType.DMA((NC,)),
            pltpu.SemaphoreType.DMA((NC,)),
            pltpu.SemaphoreType.DMA((NC,)),
            pltpu.SemaphoreType.DMA((NC,)),
        ],
        compiler_params=pltpu.CompilerParams(
            vmem_limit_bytes=100 * 1024 * 1024,
        ),
    )(partial2, resid, gamma2)


# device time: 67671 ns/iter; 1.8252x vs baseline; 1.0879x over previous
import jax
import jax.numpy as jnp
from jax import lax
from jax.experimental import pallas as pl
from jax.experimental.pallas import tpu as pltpu

M = 2048
D = 2048
HALF = M // 2
CHUNKS = [144, 144, 144, 144, 144, 144, 128, 32]
OFFS = [sum(CHUNKS[:i]) for i in range(len(CHUNKS))]
NC = len(CHUNKS)
LAG = 2
assert sum(CHUNKS) == HALF


def kernel(partial, resid, gamma):
    partial2 = partial.reshape(M, D)
    gamma2 = gamma.reshape(1, D)

    def body(
        p_hbm,
        r_hbm,
        g_ref,
        out_hbm,
        p_mine,
        r_mine,
        x_send,
        x_recv,
        y_send,
        load_p_sems,
        load_r_sem,
        store_sem,
        x_send_sems,
        x_recv_sems,
        y_send_sems,
        y_recv_sems,
    ):
        my_x = lax.axis_index("x")
        my_y = lax.axis_index("y")
        row0 = my_y * HALF
        other_row0 = (1 - my_y) * HALF

        def chunk(ref, c):
            return ref.at[pl.ds(OFFS[c], CHUNKS[c]), :]

        loads = []
        for c in range(NC):
            cp = pltpu.make_async_copy(
                p_hbm.at[pl.ds(row0 + OFFS[c], CHUNKS[c]), :],
                chunk(p_mine, c),
                load_p_sems.at[c],
            )
            cp.start()
            loads.append(cp)
        cp_r = pltpu.make_async_copy(
            r_hbm.at[pl.ds(row0, HALF), :], r_mine, load_r_sem
        )
        cp_r.start()

        barrier_sem = pltpu.get_barrier_semaphore()
        for nbr in [(1 - my_x, my_y), (my_x, 1 - my_y)]:
            pl.semaphore_signal(
                barrier_sem,
                inc=1,
                device_id=nbr,
                device_id_type=pl.DeviceIdType.MESH,
            )
        pl.semaphore_wait(barrier_sem, 2)

        x_rdmas = []
        for c in range(NC):
            loads[c].wait()
            x_send[pl.ds(OFFS[c], CHUNKS[c]), :] = p_mine[
                pl.ds(OFFS[c], CHUNKS[c]), :
            ].astype(jnp.bfloat16)
            rdma = pltpu.make_async_remote_copy(
                src_ref=chunk(x_send, c),
                dst_ref=chunk(x_recv, c),
                send_sem=x_send_sems.at[c],
                recv_sem=x_recv_sems.at[c],
                device_id=(1 - my_x, my_y),
                device_id_type=pl.DeviceIdType.MESH,
            )
            rdma.start()
            x_rdmas.append(rdma)

        cp_r.wait()

        y_rdmas = []
        stores = []

        def drain_y(c):
            y_rdmas[c].wait_recv()

        for c in range(NC):
            x_rdmas[c].wait_recv()
            sl = pl.ds(OFFS[c], CHUNKS[c])
            t = (
                p_mine[sl, :]
                + x_recv[sl, :].astype(jnp.float32)
                + r_mine[sl, :]
            )
            rms = jnp.sqrt(jnp.mean(t * t, axis=-1, keepdims=True) + 1e-6)
            y_send[sl, :] = (t / rms * g_ref[...]).astype(jnp.bfloat16)
            rdma = pltpu.make_async_remote_copy(
                src_ref=chunk(y_send, c),
                dst_ref=out_hbm.at[pl.ds(row0 + OFFS[c], CHUNKS[c]), :],
                send_sem=y_send_sems.at[c],
                recv_sem=y_recv_sems.at[c],
                device_id=(my_x, 1 - my_y),
                device_id_type=pl.DeviceIdType.MESH,
            )
            rdma.start()
            y_rdmas.append(rdma)
            cp = pltpu.make_async_copy(
                chunk(y_send, c),
                out_hbm.at[pl.ds(row0 + OFFS[c], CHUNKS[c]), :],
                store_sem,
            )
            cp.start()
            stores.append(cp)
            if c >= LAG:
                drain_y(c - LAG)

        for c in range(NC - LAG, NC):
            drain_y(c)

        for c in range(NC):
            x_rdmas[c].wait_send()
            y_rdmas[c].wait_send()
        for cp in stores:
            cp.wait()

    return pl.pallas_call(
        body,
        out_shape=jax.ShapeDtypeStruct((M, D), jnp.bfloat16),
        in_specs=[
            pl.BlockSpec(memory_space=pl.ANY),
            pl.BlockSpec(memory_space=pl.ANY),
            pl.BlockSpec(memory_space=pltpu.VMEM),
        ],
        out_specs=pl.BlockSpec(memory_space=pltpu.MemorySpace.HBM),
        scratch_shapes=[
            pltpu.VMEM((HALF, D), jnp.float32),
            pltpu.VMEM((HALF, D), jnp.float32),
            pltpu.VMEM((HALF, D), jnp.bfloat16),
            pltpu.VMEM((HALF, D), jnp.bfloat16),
            pltpu.VMEM((HALF, D), jnp.bfloat16),
            pltpu.SemaphoreType.DMA((NC,)),
            pltpu.SemaphoreType.DMA,
            pltpu.SemaphoreType.DMA,
            pltpu.SemaphoreType.DMA((NC,)),
            pltpu.SemaphoreType.DMA((NC,)),
            pltpu.SemaphoreType.DMA((NC,)),
            pltpu.SemaphoreType.DMA((NC,)),
        ],
        compiler_params=pltpu.CompilerParams(
            vmem_limit_bytes=100 * 1024 * 1024,
            collective_id=0,
        ),
    )(partial2, resid, gamma2)


# device time: 66853 ns/iter; 1.8475x vs baseline; 1.0122x over previous
import jax
import jax.numpy as jnp
from jax import lax
from jax.experimental import pallas as pl
from jax.experimental.pallas import tpu as pltpu

M = 2048
D = 2048
HALF = M // 2
CHUNKS = [128] * 8
OFFS = [sum(CHUNKS[:i]) for i in range(len(CHUNKS))]
NC = len(CHUNKS)
LAG = 2
assert sum(CHUNKS) == HALF


def kernel(partial, resid, gamma):
    partial2 = partial.reshape(M, D)
    gamma2 = gamma.reshape(1, D)

    def body(
        p_hbm,
        r_hbm,
        g_ref,
        out_hbm,
        p_mine,
        r_mine,
        x_send,
        x_recv,
        y_send,
        load_p_sems,
        load_r_sem,
        store_sem,
        x_send_sems,
        x_recv_sems,
        y_send_sems,
        y_recv_sems,
    ):
        my_x = lax.axis_index("x")
        my_y = lax.axis_index("y")
        row0 = my_y * HALF
        other_row0 = (1 - my_y) * HALF

        def chunk(ref, c):
            return ref.at[pl.ds(OFFS[c], CHUNKS[c]), :]

        loads = []
        for c in range(NC):
            cp = pltpu.make_async_copy(
                p_hbm.at[pl.ds(row0 + OFFS[c], CHUNKS[c]), :],
                chunk(p_mine, c),
                load_p_sems.at[c],
            )
            cp.start()
            loads.append(cp)
        cp_r = pltpu.make_async_copy(
            r_hbm.at[pl.ds(row0, HALF), :], r_mine, load_r_sem
        )
        cp_r.start()

        barrier_sem = pltpu.get_barrier_semaphore()
        for nbr in [(1 - my_x, my_y), (my_x, 1 - my_y)]:
            pl.semaphore_signal(
                barrier_sem,
                inc=1,
                device_id=nbr,
                device_id_type=pl.DeviceIdType.MESH,
            )
        pl.semaphore_wait(barrier_sem, 2)

        x_rdmas = []
        for c in range(NC):
            loads[c].wait()
            x_send[pl.ds(OFFS[c], CHUNKS[c]), :] = p_mine[
                pl.ds(OFFS[c], CHUNKS[c]), :
            ].astype(jnp.bfloat16)
            rdma = pltpu.make_async_remote_copy(
                src_ref=chunk(x_send, c),
                dst_ref=chunk(x_recv, c),
                send_sem=x_send_sems.at[c],
                recv_sem=x_recv_sems.at[c],
                device_id=(1 - my_x, my_y),
                device_id_type=pl.DeviceIdType.MESH,
            )
            rdma.start()
            x_rdmas.append(rdma)

        cp_r.wait()

        y_rdmas = []
        stores = []

        def drain_y(c):
            y_rdmas[c].wait_recv()

        for c in range(NC):
            x_rdmas[c].wait_recv()
            sl = pl.ds(OFFS[c], CHUNKS[c])
            t = (
                p_mine[sl, :]
                + x_recv[sl, :].astype(jnp.float32)
                + r_mine[sl, :]
            )
            rms = jnp.sqrt(jnp.mean(t * t, axis=-1, keepdims=True) + 1e-6)
            y_send[sl, :] = (t / rms * g_ref[...]).astype(jnp.bfloat16)
            rdma = pltpu.make_async_remote_copy(
                src_ref=chunk(y_send, c),
                dst_ref=out_hbm.at[pl.ds(row0 + OFFS[c], CHUNKS[c]), :],
                send_sem=y_send_sems.at[c],
                recv_sem=y_recv_sems.at[c],
                device_id=(my_x, 1 - my_y),
                device_id_type=pl.DeviceIdType.MESH,
            )
            rdma.start()
            y_rdmas.append(rdma)
            cp = pltpu.make_async_copy(
                chunk(y_send, c),
                out_hbm.at[pl.ds(row0 + OFFS[c], CHUNKS[c]), :],
                store_sem,
            )
            cp.start()
            stores.append(cp)
            if c >= LAG:
                drain_y(c - LAG)

        for c in range(NC - LAG, NC):
            drain_y(c)

        for c in range(NC):
            x_rdmas[c].wait_send()
            y_rdmas[c].wait_send()
        for cp in stores:
            cp.wait()

    return pl.pallas_call(
        body,
        out_shape=jax.ShapeDtypeStruct((M, D), jnp.bfloat16),
        in_specs=[
            pl.BlockSpec(memory_space=pl.ANY),
            pl.BlockSpec(memory_space=pl.ANY),
            pl.BlockSpec(memory_space=pltpu.VMEM),
        ],
        out_specs=pl.BlockSpec(memory_space=pltpu.MemorySpace.HBM),
        scratch_shapes=[
            pltpu.VMEM((HALF, D), jnp.float32),
            pltpu.VMEM((HALF, D), jnp.float32),
            pltpu.VMEM((HALF, D), jnp.bfloat16),
            pltpu.VMEM((HALF, D), jnp.bfloat16),
            pltpu.VMEM((HALF, D), jnp.bfloat16),
            pltpu.SemaphoreType.DMA((NC,)),
            pltpu.SemaphoreType.DMA,
            pltpu.SemaphoreType.DMA,
            pltpu.SemaphoreType.DMA((NC,)),
            pltpu.SemaphoreType.DMA((NC,)),
            pltpu.SemaphoreType.DMA((NC,)),
            pltpu.SemaphoreType.DMA((NC,)),
        ],
        compiler_params=pltpu.CompilerParams(
            vmem_limit_bytes=100 * 1024 * 1024,
            collective_id=0,
        ),
    )(partial2, resid, gamma2)
